# Initial kernel scaffold; baseline (speedup 1.0000x reference)
import jax
import jax.numpy as jnp
from jax.experimental import pallas as pl


def kernel(code_tokens, embedding_table):
    raise NotImplementedError("write your pallas kernel here")


# speedup vs baseline: 1.4891x; 1.4891x over previous
import functools

import jax
import jax.numpy as jnp
from jax import lax
from jax.experimental import pallas as pl
from jax.experimental.pallas import tpu as pltpu
from jax.experimental.pallas import tpu_sc as plsc

VOCAB = 1000000
EMBED_DIM = 32
B_TOTAL = 4096 * 200

_info = plsc.get_sparse_core_info()
_NC, _NS = _info.num_cores, _info.num_subcores
_NW = _NC * _NS
_B_PER_W = B_TOTAL // _NW
_CHUNK = 2560
_NCHUNK = _B_PER_W // _CHUNK


def _make_gather():
    mesh = plsc.VectorSubcoreMesh(core_axis_name="c", subcore_axis_name="s")

    @functools.partial(
        pl.kernel,
        mesh=mesh,
        out_type=jax.ShapeDtypeStruct((B_TOTAL, EMBED_DIM), jnp.float32),
        scratch_types=[
            pltpu.VMEM((_CHUNK,), jnp.int32),
            pltpu.VMEM((_CHUNK, EMBED_DIM), jnp.float32),
            pltpu.SemaphoreType.DMA,
        ],
        compiler_params=pltpu.CompilerParams(use_tc_tiling_on_sc=False),
    )
    def gather_kernel(table_hbm, idx_hbm, out_hbm, idx_v, rows_v, sem):
        wid = lax.axis_index("s") * _NC + lax.axis_index("c")
        base = wid * _B_PER_W
        for j in range(_NCHUNK):
            off = base + j * _CHUNK
            pltpu.sync_copy(idx_hbm.at[pl.ds(off, _CHUNK)], idx_v)
            pltpu.async_copy(table_hbm.at[idx_v], rows_v, sem).wait()
            pltpu.sync_copy(rows_v, out_hbm.at[pl.ds(off, _CHUNK)])

    return gather_kernel


_gather = _make_gather()


@jax.jit
def kernel(code_tokens, embedding_table):
    idx = code_tokens.reshape(B_TOTAL).astype(jnp.int32)
    out = _gather(embedding_table, idx)
    return out.reshape(code_tokens.shape[0], code_tokens.shape[1], EMBED_DIM)


# speedup vs baseline: 1.4936x; 1.0030x over previous
import functools

import jax
import jax.numpy as jnp
from jax import lax
from jax.experimental import pallas as pl
from jax.experimental.pallas import tpu as pltpu
from jax.experimental.pallas import tpu_sc as plsc

VOCAB = 1000000
EMBED_DIM = 32
B_TOTAL = 4096 * 200

_info = plsc.get_sparse_core_info()
_NC, _NS = _info.num_cores, _info.num_subcores
_NW = _NC * _NS
_B_PER_W = B_TOTAL // _NW
_CHUNK = 1600
_NCHUNK = _B_PER_W // _CHUNK


def _make_gather():
    mesh = plsc.VectorSubcoreMesh(core_axis_name="c", subcore_axis_name="s")

    @functools.partial(
        pl.kernel,
        mesh=mesh,
        out_type=jax.ShapeDtypeStruct((B_TOTAL, EMBED_DIM), jnp.float32),
        scratch_types=[
            pltpu.VMEM((2, _CHUNK), jnp.int32),
            pltpu.VMEM((2, _CHUNK, EMBED_DIM), jnp.float32),
            [pltpu.SemaphoreType.DMA] * 2,
            [pltpu.SemaphoreType.DMA] * 2,
            [pltpu.SemaphoreType.DMA] * 2,
        ],
        compiler_params=pltpu.CompilerParams(use_tc_tiling_on_sc=False),
    )
    def gather_kernel(table_hbm, idx_hbm, out_hbm, idx_v, rows_v, si, sg, ss):
        wid = lax.axis_index("s") * _NC + lax.axis_index("c")
        base = wid * _B_PER_W

        def idx_copy(j, b):
            return pltpu.make_async_copy(
                idx_hbm.at[pl.ds(base + j * _CHUNK, _CHUNK)], idx_v.at[b], si[b]
            )

        def gather_copy(j, b):
            return pltpu.make_async_copy(table_hbm.at[idx_v.at[b]], rows_v.at[b], sg[b])

        def store_copy(j, b):
            return pltpu.make_async_copy(
                rows_v.at[b], out_hbm.at[pl.ds(base + j * _CHUNK, _CHUNK)], ss[b]
            )

        idx_copy(0, 0).start()
        idx_copy(1, 1).start()
        for j in range(_NCHUNK):
            b = j % 2
            idx_copy(j, b).wait()
            if j >= 2:
                store_copy(j - 2, b).wait()
            g = gather_copy(j, b)
            g.start()
            g.wait()
            if j + 2 < _NCHUNK:
                idx_copy(j + 2, b).start()
            store_copy(j, b).start()
        store_copy(_NCHUNK - 2, (_NCHUNK - 2) % 2).wait()
        store_copy(_NCHUNK - 1, (_NCHUNK - 1) % 2).wait()

    return gather_kernel


_gather = _make_gather()


@jax.jit
def kernel(code_tokens, embedding_table):
    idx = code_tokens.reshape(B_TOTAL).astype(jnp.int32)
    out = _gather(embedding_table, idx)
    return out.reshape(code_tokens.shape[0], code_tokens.shape[1], EMBED_DIM)
